# baseline (speedup 1.0000x reference)
import functools

import jax
import jax.numpy as jnp
from jax import lax
from jax.experimental import pallas as pl
from jax.experimental.pallas import tpu as pltpu
from jax.experimental.pallas import tpu_sc as plsc

_B, _S, _D = 4096, 200, 64
_V = 1000000
_SCALE = float(_D) ** 0.5
_NC, _NS, _L = 2, 16, 16
_NW = _NC * _NS
_ROWS = _B * _S
_RPW = _ROWS // _NW
_C = 256
_NCHUNK = _RPW // _C


def _emb_body(ids_hbm, table_hbm, out_hbm, idx_v, idxh_v, rows_v, out_v,
              sem):
    wid = lax.axis_index("s") * _NC + lax.axis_index("c")
    base = wid * _RPW

    def chunk(g, carry):
        off = base + g * _C
        pltpu.sync_copy(ids_hbm.at[pl.ds(off, _C)], idx_v)

        def halve(i, c2):
            sl = pl.ds(i * _L, _L)
            idxh_v[sl] = lax.shift_right_logical(idx_v[sl], 1)
            return c2

        lax.fori_loop(0, _C // _L, halve, 0)
        pltpu.async_copy(table_hbm.at[idxh_v], rows_v, sem).wait()

        lanes = lax.iota(jnp.int32, _L)

        def pick_row(r, c2):
            rsplat = jnp.full((_L,), r, jnp.int32)
            hv = plsc.load_gather(idx_v, [rsplat])
            hbase = (hv & 1) * _D
            for j in range(_D // _L):
                col = hbase + (j * _L) + lanes
                val = plsc.load_gather(rows_v, [rsplat, col])
                out_v[r, pl.ds(j * _L, _L)] = val * _SCALE
            return c2

        lax.fori_loop(0, _C, pick_row, 0)
        pltpu.sync_copy(out_v, out_hbm.at[pl.ds(off, _C)])
        return carry

    lax.fori_loop(0, _NCHUNK, chunk, 0)


def kernel(input_ids, weight):
    ids = input_ids.reshape(_ROWS)
    table = weight.reshape(_V // 2, 2 * _D)
    mesh = plsc.VectorSubcoreMesh(core_axis_name="c", subcore_axis_name="s")
    run = functools.partial(
        pl.kernel,
        mesh=mesh,
        compiler_params=pltpu.CompilerParams(
            use_tc_tiling_on_sc=True, needs_layout_passes=False),
        out_type=jax.ShapeDtypeStruct((_ROWS, _D), jnp.float32),
        scratch_types=[
            pltpu.VMEM((_C,), jnp.int32),
            pltpu.VMEM((_C,), jnp.int32),
            pltpu.VMEM((_C, 2 * _D), jnp.float32),
            pltpu.VMEM((_C, _D), jnp.float32),
            pltpu.SemaphoreType.DMA,
        ],
    )(_emb_body)
    out = run(ids, table)
    return out.reshape(_B, _S, _D)


# speedup vs baseline: 1.4346x; 1.4346x over previous
import functools

import jax
import jax.numpy as jnp
from jax import lax
from jax.experimental import pallas as pl
from jax.experimental.pallas import tpu as pltpu
from jax.experimental.pallas import tpu_sc as plsc

_B, _S, _D = 4096, 200, 64
_V = 1000000
_SCALE = float(_D) ** 0.5
_NC, _NS, _L = 2, 16, 16
_NW = _NC * _NS
_ROWS = _B * _S
_RPW = _ROWS // _NW
_C = 256
_NCHUNK = _RPW // _C
_RP = 8000


def _prep_body(w_ref, o_ref):
    o_ref[:, 0:_D] = w_ref[...] * _SCALE


def _prep_table(weight):
    return pl.pallas_call(
        _prep_body,
        grid=(_V // _RP,),
        in_specs=[pl.BlockSpec((_RP, _D), lambda i: (i, 0))],
        out_specs=pl.BlockSpec((_RP, 2 * _D), lambda i: (i, 0)),
        out_shape=jax.ShapeDtypeStruct((_V, 2 * _D), jnp.float32),
    )(weight)


def _emb_body(ids_hbm, table_hbm, out_hbm, idx_v, rows_v, out_v, sem):
    wid = lax.axis_index("s") * _NC + lax.axis_index("c")
    base = wid * _RPW

    def chunk(g, carry):
        off = base + g * _C
        pltpu.sync_copy(ids_hbm.at[pl.ds(off, _C)], idx_v)
        pltpu.async_copy(table_hbm.at[idx_v], rows_v, sem).wait()

        def extract_row(r, c2):
            for j in range(_D // _L):
                sl = pl.ds(j * _L, _L)
                out_v[r, sl] = rows_v[r, sl]
            return c2

        lax.fori_loop(0, _C, extract_row, 0)
        pltpu.sync_copy(out_v, out_hbm.at[pl.ds(off, _C)])
        return carry

    lax.fori_loop(0, _NCHUNK, chunk, 0)


def kernel(input_ids, weight):
    ids = input_ids.reshape(_ROWS)
    table = _prep_table(weight)
    mesh = plsc.VectorSubcoreMesh(core_axis_name="c", subcore_axis_name="s")
    run = functools.partial(
        pl.kernel,
        mesh=mesh,
        compiler_params=pltpu.CompilerParams(
            use_tc_tiling_on_sc=True, needs_layout_passes=False),
        out_type=jax.ShapeDtypeStruct((_ROWS, _D), jnp.float32),
        scratch_types=[
            pltpu.VMEM((_C,), jnp.int32),
            pltpu.VMEM((_C, 2 * _D), jnp.float32),
            pltpu.VMEM((_C, _D), jnp.float32),
            pltpu.SemaphoreType.DMA,
        ],
    )(_emb_body)
    out = run(ids, table)
    return out.reshape(_B, _S, _D)


# speedup vs baseline: 1.9574x; 1.3644x over previous
import functools

import jax
import jax.numpy as jnp
from jax import lax
from jax.experimental import pallas as pl
from jax.experimental.pallas import tpu as pltpu
from jax.experimental.pallas import tpu_sc as plsc

_B, _S, _D = 4096, 200, 64
_V = 1000000
_SCALE = float(_D) ** 0.5
_NC, _NS, _L = 2, 16, 16
_NW = _NC * _NS
_ROWS = _B * _S
_RPW = _ROWS // _NW
_C = 200
_NCHUNK = _RPW // _C
_BV = 2048


def _prep_body(wt_ref, o_ref):
    o_ref[:, 0:_D] = wt_ref[...].T * _SCALE


def _prep_table(weight_t):
    return pl.pallas_call(
        _prep_body,
        grid=(pl.cdiv(_V, _BV),),
        in_specs=[pl.BlockSpec((_D, _BV), lambda i: (0, i))],
        out_specs=pl.BlockSpec((_BV, 2 * _D), lambda i: (i, 0)),
        out_shape=jax.ShapeDtypeStruct((_V, 2 * _D), jnp.float32),
    )(weight_t)


def _emb_body(ids_hbm, table_hbm, out_hbm,
              idx_a, idx_b, rows_a, rows_b, out_a, out_b, sem_a, sem_b):
    wid = lax.axis_index("s") * _NC + lax.axis_index("c")
    base = wid * _RPW

    def stage(g, idx_v, rows_v, sem):
        off = base + g * _C
        pltpu.sync_copy(ids_hbm.at[pl.ds(off, _C)], idx_v)
        pltpu.async_copy(table_hbm.at[idx_v], rows_v, sem)

    def drain(g, rows_v, out_v):
        def extract_row(r, c2):
            for j in range(_D // _L):
                sl = pl.ds(j * _L, _L)
                out_v[r, sl] = rows_v[r, sl]
            return c2

        lax.fori_loop(0, _C, extract_row, 0)
        off = base + g * _C
        pltpu.sync_copy(out_v, out_hbm.at[pl.ds(off, _C)])

    stage(0, idx_a, rows_a, sem_a)

    def pair(k, carry):
        g0 = 2 * k
        stage(g0 + 1, idx_b, rows_b, sem_b)
        pltpu.make_async_copy(table_hbm.at[idx_a], rows_a, sem_a).wait()
        drain(g0, rows_a, out_a)

        @pl.when(k + 1 < _NCHUNK // 2)
        def _():
            stage(g0 + 2, idx_a, rows_a, sem_a)

        pltpu.make_async_copy(table_hbm.at[idx_b], rows_b, sem_b).wait()
        drain(g0 + 1, rows_b, out_b)
        return carry

    lax.fori_loop(0, _NCHUNK // 2, pair, 0)


def kernel(input_ids, weight):
    ids = input_ids.reshape(_ROWS)
    table = _prep_table(weight.T)
    mesh = plsc.VectorSubcoreMesh(core_axis_name="c", subcore_axis_name="s")
    run = functools.partial(
        pl.kernel,
        mesh=mesh,
        compiler_params=pltpu.CompilerParams(
            use_tc_tiling_on_sc=True, needs_layout_passes=False),
        out_type=jax.ShapeDtypeStruct((_ROWS, _D), jnp.float32),
        scratch_types=[
            pltpu.VMEM((_C,), jnp.int32),
            pltpu.VMEM((_C,), jnp.int32),
            pltpu.VMEM((_C, 2 * _D), jnp.float32),
            pltpu.VMEM((_C, 2 * _D), jnp.float32),
            pltpu.VMEM((_C, _D), jnp.float32),
            pltpu.VMEM((_C, _D), jnp.float32),
            pltpu.SemaphoreType.DMA,
            pltpu.SemaphoreType.DMA,
        ],
    )(_emb_body)
    out = run(ids, table)
    return out.reshape(_B, _S, _D)
